# baseline (speedup 1.0000x reference)
import jax
import jax.numpy as jnp
from jax.experimental import pallas as pl
from jax.experimental.pallas import tpu as pltpu


def _fused_head_kernel(n_steps, x_ref, wt_ref, b_ref, lbl_ref,
                       acc_ref, pooled_ref):
    i = pl.program_id(0)
    partial = jnp.sum(x_ref[...], axis=0)

    @pl.when(i == 0)
    def _first():
        pooled_ref[...] = partial

    @pl.when(i > 0)
    def _rest():
        pooled_ref[...] += partial

    @pl.when(i == n_steps - 1)
    def _head():
        hw_total = n_steps * x_ref.shape[0]
        pooled = pooled_ref[...] * (1.0 / hw_total)

        logits = jax.lax.dot_general(
            pooled, wt_ref[...], (((1,), (1,)), ((), ())),
            preferred_element_type=jnp.float32) + b_ref[...]

        m = jnp.max(logits, axis=1, keepdims=True)
        lse = m + jnp.log(jnp.sum(jnp.exp(logits - m), axis=1,
                                  keepdims=True))
        tn, k = logits.shape
        cls_iota = jax.lax.broadcasted_iota(jnp.int32, (tn, k), 1)
        lbl = lbl_ref[...]
        picked = jnp.sum(jnp.where(cls_iota == lbl, logits, 0.0),
                         axis=1, keepdims=True)
        loss = lse - picked

        n_greater = jnp.sum((logits > picked).astype(jnp.float32),
                            axis=1, keepdims=True)
        n_eq_before = jnp.sum(((logits == picked) & (cls_iota < lbl))
                              .astype(jnp.float32), axis=1, keepdims=True)
        rank = n_greater + n_eq_before
        hit1 = (rank < 1.0).astype(jnp.float32)
        hit5 = (rank < 5.0).astype(jnp.float32)

        lane = jax.lax.broadcasted_iota(jnp.int32, (1, 128), 1)
        acc_ref[...] = (jnp.where(lane == 0, jnp.sum(loss), 0.0) +
                        jnp.where(lane == 1, jnp.sum(hit1), 0.0) +
                        jnp.where(lane == 2, jnp.sum(hit5), 0.0))


def kernel(x, w, b, gt_label):
    import functools

    N, C, H, W = x.shape
    K = w.shape[1]
    HW = H * W

    xt = jnp.transpose(x, (2, 3, 0, 1)).reshape(HW, N, C)
    wt = jnp.transpose(w)
    b2 = b.reshape(1, K)
    lbl2 = gt_label.astype(jnp.int32).reshape(N, 1)

    HW_TILE = H
    n_steps = pl.cdiv(HW, HW_TILE)

    acc = pl.pallas_call(
        functools.partial(_fused_head_kernel, n_steps),
        out_shape=jax.ShapeDtypeStruct((1, 128), jnp.float32),
        grid=(n_steps,),
        in_specs=[
            pl.BlockSpec((HW_TILE, N, C), lambda i: (i, 0, 0)),
            pl.BlockSpec((K, C), lambda i: (0, 0)),
            pl.BlockSpec((1, K), lambda i: (0, 0)),
            pl.BlockSpec((N, 1), lambda i: (0, 0)),
        ],
        out_specs=pl.BlockSpec((1, 128), lambda i: (0, 0)),
        scratch_shapes=[pltpu.VMEM((N, C), jnp.float32)],
        compiler_params=pltpu.CompilerParams(
            dimension_semantics=("arbitrary",),
            vmem_limit_bytes=48 * 1024 * 1024,
        ),
    )(xt, wt, b2, lbl2)

    inv_n = 1.0 / N
    return {
        "loss": acc[0, 0] * inv_n,
        "accuracy": {
            "top-1": acc[0, 1] * (100.0 * inv_n),
            "top-5": acc[0, 2] * (100.0 * inv_n),
        },
    }


# speedup vs baseline: 1.0163x; 1.0163x over previous
import jax
import jax.numpy as jnp
from jax.experimental import pallas as pl
from jax.experimental.pallas import tpu as pltpu


def _fused_head_kernel(x_ref, wt_ref, b_ref, lbl_ref, acc_ref):
    x = x_ref[...]
    hw = x.shape[0]
    pooled = jnp.sum(x, axis=0) * (1.0 / hw)

    logits = jax.lax.dot_general(
        pooled, wt_ref[...], (((1,), (1,)), ((), ())),
        preferred_element_type=jnp.float32) + b_ref[...]

    m = jnp.max(logits, axis=1, keepdims=True)
    lse = m + jnp.log(jnp.sum(jnp.exp(logits - m), axis=1, keepdims=True))
    tn, k = logits.shape
    cls_iota = jax.lax.broadcasted_iota(jnp.int32, (tn, k), 1)
    lbl = lbl_ref[...]
    picked = jnp.sum(jnp.where(cls_iota == lbl, logits, 0.0),
                     axis=1, keepdims=True)
    loss = lse - picked

    n_greater = jnp.sum((logits > picked).astype(jnp.float32),
                        axis=1, keepdims=True)
    n_eq_before = jnp.sum(((logits == picked) & (cls_iota < lbl))
                          .astype(jnp.float32), axis=1, keepdims=True)
    rank = n_greater + n_eq_before
    hit1 = (rank < 1.0).astype(jnp.float32)
    hit5 = (rank < 5.0).astype(jnp.float32)

    lane = jax.lax.broadcasted_iota(jnp.int32, (1, 128), 1)
    tile_sums = (jnp.where(lane == 0, jnp.sum(loss), 0.0) +
                 jnp.where(lane == 1, jnp.sum(hit1), 0.0) +
                 jnp.where(lane == 2, jnp.sum(hit5), 0.0))

    @pl.when(pl.program_id(0) == 0)
    def _init():
        acc_ref[...] = jnp.zeros_like(acc_ref)

    acc_ref[...] += tile_sums


def kernel(x, w, b, gt_label):
    N, C, H, W = x.shape
    K = w.shape[1]
    HW = H * W

    xt = jnp.transpose(x, (2, 3, 0, 1)).reshape(HW, N, C)
    wt = jnp.transpose(w)
    b2 = b.reshape(1, K)
    lbl2 = gt_label.astype(jnp.int32).reshape(N, 1)

    TILE_N = min(N, 32)
    grid = (pl.cdiv(N, TILE_N),)

    acc = pl.pallas_call(
        _fused_head_kernel,
        out_shape=jax.ShapeDtypeStruct((1, 128), jnp.float32),
        grid=grid,
        in_specs=[
            pl.BlockSpec((HW, TILE_N, C), lambda i: (0, i, 0)),
            pl.BlockSpec((K, C), lambda i: (0, 0)),
            pl.BlockSpec((1, K), lambda i: (0, 0)),
            pl.BlockSpec((TILE_N, 1), lambda i: (i, 0)),
        ],
        out_specs=pl.BlockSpec((1, 128), lambda i: (0, 0)),
        compiler_params=pltpu.CompilerParams(
            dimension_semantics=("arbitrary",),
            vmem_limit_bytes=48 * 1024 * 1024,
        ),
    )(xt, wt, b2, lbl2)

    inv_n = 1.0 / N
    return {
        "loss": acc[0, 0] * inv_n,
        "accuracy": {
            "top-1": acc[0, 1] * (100.0 * inv_n),
            "top-5": acc[0, 2] * (100.0 * inv_n),
        },
    }
